# baseline (speedup 1.0000x reference)
import functools
import math

import jax
import jax.numpy as jnp
from jax import lax
from jax.experimental import pallas as pl
from jax.experimental.pallas import tpu as pltpu
from jax.experimental.pallas import tpu_sc as plsc

_NC = 2
_NS = 16
_NW = _NC * _NS
_LANES = 16
_GA = 104
_NBUF = 6
_LOOK = 4


def _sc_embed(idx_a, idx_b, token_table, pos_table, *,
              batch, seq, dim, scale):
  rpw = batch // _NW
  nu = 2 * rpw
  nq = (nu - _LOOK) // _NBUF
  tail = nu - nq * _NBUF
  gsz = (_GA, seq - _GA)
  goff = (0, _GA)
  mesh = plsc.VectorSubcoreMesh(
      core_axis_name="c", subcore_axis_name="s",
      num_cores=_NC, num_subcores=_NS)

  @functools.partial(
      pl.kernel,
      mesh=mesh,
      out_type=jax.ShapeDtypeStruct((batch, seq, dim), jnp.float32),
      scratch_types=(
          [pltpu.VMEM((rpw, gsz[0]), jnp.int32),
           pltpu.VMEM((rpw, gsz[1]), jnp.int32),
           pltpu.VMEM((_NBUF, _GA, dim), jnp.float32),
           pltpu.VMEM((seq, dim), jnp.float32)]
          + [pltpu.SemaphoreType.DMA] * (2 * _NBUF + 1)
      ),
  )
  def k(ia_hbm, ib_hbm, table_hbm, pos_hbm, out_hbm,
        ia_v, ib_v, rows_v, pos_v, *sems):
    sem_g = sems[:_NBUF]
    sem_w = sems[_NBUF:2 * _NBUF]
    sem_p = sems[2 * _NBUF]
    idx_v = (ia_v, ib_v)
    wid = lax.axis_index("s") * _NC + lax.axis_index("c")
    rbase = wid * rpw
    pltpu.sync_copy(ia_hbm.at[pl.ds(rbase, rpw)], ia_v)
    pltpu.sync_copy(ib_hbm.at[pl.ds(rbase, rpw)], ib_v)

    def gather(row_l, half, b):
      pltpu.async_copy(
          table_hbm.at[idx_v[half].at[row_l]],
          rows_v.at[b, pl.ds(0, gsz[half])], sem_g[b])

    def wait_gather(half, b):
      pltpu.make_async_copy(
          table_hbm.at[idx_v[half].at[0]],
          rows_v.at[b, pl.ds(0, gsz[half])], sem_g[b]).wait()

    def wait_write(half, b):
      pltpu.make_async_copy(
          rows_v.at[b, pl.ds(0, gsz[half])],
          out_hbm.at[0, pl.ds(goff[half], gsz[half])], sem_w[b]).wait()

    def fma_write(u_row, half, b):
      def fma(l, c, _b=b, _ph=goff[half]):
        for d in range(dim // _LANES):
          sl = pl.ds(d * _LANES, _LANES)
          rows_v[_b, l, sl] = rows_v[_b, l, sl] * scale + pos_v[_ph + l, sl]
        return c
      lax.fori_loop(0, gsz[half], fma, 0)
      pltpu.async_copy(
          rows_v.at[b, pl.ds(0, gsz[half])],
          out_hbm.at[u_row, pl.ds(goff[half], gsz[half])], sem_w[b])

    for u in range(_LOOK):
      gather(u // 2, u % 2, u)
    cpp = pltpu.async_copy(pos_hbm, pos_v, sem_p)
    cpp.wait()

    def step(q, carry):
      for b in range(_NBUF):
        u = q * _NBUF + b
        row_l = q * (_NBUF // 2) + b // 2
        half = b % 2
        wait_gather(half, b)

        bn = (b + _LOOK) % _NBUF
        if b < _NBUF - _LOOK:
          @pl.when(q > 0)
          def _():
            wait_write(half, bn)
        else:
          wait_write(half, bn)
        gather(row_l + _LOOK // 2, half, bn)

        fma_write(rbase + row_l, half, b)
      return carry

    lax.fori_loop(0, nq, step, 0)

    for t in range(tail):
      u = nq * _NBUF + t
      b = u % _NBUF
      wait_gather(u % 2, b)
      fma_write(rbase + u // 2, u % 2, b)

    for u in range(nu - _NBUF, nu):
      wait_write(u % 2, u % _NBUF)

  return k(idx_a, idx_b, token_table, pos_table)


def kernel(inputs, token_table, pos_table):
  batch, seq = inputs.shape
  vocab, dim = token_table.shape
  scale = float(math.sqrt(dim))
  return _sc_embed(inputs[:, :_GA], inputs[:, _GA:], token_table, pos_table,
                   batch=batch, seq=seq, dim=dim, scale=scale)
